# Initial kernel scaffold; baseline (speedup 1.0000x reference)
import jax
import jax.numpy as jnp
from jax.experimental import pallas as pl


def kernel(x, weight, bias, cluster_centers, temp):
    raise NotImplementedError("write your pallas kernel here")


# speedup vs baseline: 4.7941x; 4.7941x over previous
import functools

import jax
import jax.numpy as jnp
from jax.experimental import pallas as pl
from jax.experimental.pallas import tpu as pltpu

_C = 96
_H = 224
_NC = 512
_PS = 864
_L = _H * _H

_RT = 1792
_LT = 1792


def _unfold_body(xp_ref, o_ref):
    xp = xp_ref[0]
    for ki in range(3):
        for kj in range(3):
            o_ref[0, ki * 3 + kj] = xp[ki:ki + _H, kj:kj + _H]


def _proto_body(scal_ref, z_ref, c_ref, o_ref):
    tempv = scal_ref[0]
    alpha = scal_ref[1]
    beta = scal_ref[2]
    f = z_ref[...]
    c = c_ref[...]
    g = jax.lax.dot_general(f, c, (((1,), (1,)), ((), ())),
                            preferred_element_type=jnp.float32)
    f2 = jnp.sum(f * f, axis=1, keepdims=True)
    c2 = jnp.sum(c * c, axis=1)[None, :]
    d2 = f2 + c2 - 2.0 * g
    d = jnp.sqrt(jnp.maximum(d2, 1e-12))
    neg = -d * tempv
    m = jnp.max(neg, axis=1, keepdims=True)
    e = jnp.exp(neg - m)
    s = e / jnp.sum(e, axis=1, keepdims=True)
    t = jax.lax.dot_general(s, c, (((1,), (0,)), ((), ())),
                            preferred_element_type=jnp.float32)
    o_ref[...] = alpha * t + beta * f


def _conv_body(z_ref, w_ref, b_ref, o_ref):
    i = pl.program_id(0)
    z = z_ref[...]
    ch = jax.lax.broadcasted_iota(jnp.int32, z.shape, 0)
    l = jax.lax.broadcasted_iota(jnp.int32, z.shape, 1) + i * _LT
    top = ((ch % 9) < 3) & (l < _H)
    left = ((ch % 3) == 0) & ((l % _H) == 0)
    zm = jnp.where(top | left, 0.0, z)
    o_ref[...] = jax.lax.dot_general(w_ref[...], zm, (((1,), (0,)), ((), ())),
                                     preferred_element_type=jnp.float32) + b_ref[...]


def kernel(x, weight, bias, cluster_centers, temp):
    xp = jnp.pad(x[0], ((0, 0), (1, 1), (1, 1)))

    z4 = pl.pallas_call(
        _unfold_body,
        grid=(_C,),
        in_specs=[pl.BlockSpec((1, _H + 2, _H + 2), lambda i: (i, 0, 0))],
        out_specs=pl.BlockSpec((1, 9, _H, _H), lambda i: (i, 0, 0, 0)),
        out_shape=jax.ShapeDtypeStruct((_C, 9, _H, _H), jnp.float32),
        compiler_params=pltpu.CompilerParams(
            dimension_semantics=("parallel",)),
    )(xp)
    zf = z4.reshape(_L, _PS)

    tempf = jnp.asarray(temp, jnp.float32)
    scal = jnp.stack([tempf, tempf / (tempf + 1.0), 1.0 / (tempf + 1.0), tempf])

    f2 = pl.pallas_call(
        _proto_body,
        grid=(_L // _RT,),
        in_specs=[
            pl.BlockSpec(memory_space=pltpu.SMEM),
            pl.BlockSpec((_RT, _PS), lambda i: (i, 0)),
            pl.BlockSpec((_NC, _PS), lambda i: (0, 0)),
        ],
        out_specs=pl.BlockSpec((_RT, _PS), lambda i: (i, 0)),
        out_shape=jax.ShapeDtypeStruct((_L, _PS), jnp.float32),
        compiler_params=pltpu.CompilerParams(
            dimension_semantics=("parallel",)),
    )(scal, zf, cluster_centers)

    z2 = f2.reshape(_PS, _L)
    w2 = weight.reshape(_C, _PS)
    b2 = bias.reshape(_C, 1)

    out = pl.pallas_call(
        _conv_body,
        grid=(_L // _LT,),
        in_specs=[
            pl.BlockSpec((_PS, _LT), lambda i: (0, i)),
            pl.BlockSpec((_C, _PS), lambda i: (0, 0)),
            pl.BlockSpec((_C, 1), lambda i: (0, 0)),
        ],
        out_specs=pl.BlockSpec((_C, _LT), lambda i: (0, i)),
        out_shape=jax.ShapeDtypeStruct((_C, _L), jnp.float32),
        compiler_params=pltpu.CompilerParams(
            dimension_semantics=("parallel",)),
    )(z2, w2, b2)

    return out.reshape(1, _C, _H, _H)
